# baseline (speedup 1.0000x reference)
import functools

import jax
import jax.numpy as jnp
from jax import lax
from jax.experimental import pallas as pl
from jax.experimental.pallas import tpu as pltpu
from jax.experimental.pallas import tpu_sc as plsc

B, L, D = 16, 2048, 1024
LANES = 16
HALF = D // 2
NVEC = HALF // LANES
CH = 64


def _body(xs_hbm, mi_hbm, mf_hbm, out_hbm, lbuf, fbuf, buf, obuf):
    c = lax.axis_index("c")
    s = lax.axis_index("s")
    b = c * (B // 2) + s // 2
    col0 = (s % 2) * HALF

    pltpu.sync_copy(mi_hbm.at[c, s], lbuf)
    pltpu.sync_copy(mf_hbm.at[c, s], fbuf)
    l = lbuf[...][0]
    inv = fbuf[...][0]
    nch = (l + CH - 1) // CH

    def chunk_body(i, accs):
        t0 = i * CH
        t0c = jnp.minimum(t0, L - CH)
        off = t0 - t0c
        nv = jnp.minimum(CH, l - t0)
        pltpu.sync_copy(xs_hbm.at[b, pl.ds(t0c, CH), pl.ds(col0, HALF)], buf)

        def row_body(r, a):
            row = off + r
            return tuple(
                a[v] + buf[row, pl.ds(v * LANES, LANES)] for v in range(NVEC)
            )

        return lax.fori_loop(0, nv, row_body, accs)

    accs = tuple(jnp.zeros((LANES,), jnp.float32) for _ in range(NVEC))
    accs = lax.fori_loop(0, nch, chunk_body, accs)
    for v in range(NVEC):
        obuf[pl.ds(v * LANES, LANES)] = accs[v] * inv
    pltpu.sync_copy(obuf, out_hbm.at[b, pl.ds(col0, HALF)])


@jax.jit
def _mean_pool(xs, mi, mf):
    kern = pl.kernel(
        _body,
        out_type=jax.ShapeDtypeStruct((B, D), jnp.float32),
        mesh=plsc.VectorSubcoreMesh(core_axis_name="c", subcore_axis_name="s"),
        scratch_types=[
            pltpu.VMEM((LANES,), jnp.int32),
            pltpu.VMEM((LANES,), jnp.float32),
            pltpu.VMEM((CH, HALF), jnp.float32),
            pltpu.VMEM((HALF,), jnp.float32),
        ],
    )
    return kern(xs, mi, mf)


def kernel(xs, xs_len):
    lens = xs_len.astype(jnp.int32)
    inv = 1.0 / lens.astype(jnp.float32)
    cc = jnp.arange(2)[:, None]
    ss = jnp.arange(16)[None, :]
    bmap = cc * (B // 2) + ss // 2
    mi = jnp.broadcast_to(lens[bmap][:, :, None], (2, 16, LANES))
    mf = jnp.broadcast_to(inv[bmap][:, :, None], (2, 16, LANES))
    return _mean_pool(xs, mi, mf)


# speedup vs baseline: 1.5679x; 1.5679x over previous
import functools

import jax
import jax.numpy as jnp
from jax import lax
from jax.experimental import pallas as pl
from jax.experimental.pallas import tpu as pltpu
from jax.experimental.pallas import tpu_sc as plsc

B, L, D = 16, 2048, 1024
LANES = 16
HALF = D // 2
NVEC = HALF // LANES
CH = 64


def _body(xs_hbm, mi_hbm, mf_hbm, out_hbm, lbuf, fbuf, buf0, buf1, obuf,
          sem0, sem1):
    c = lax.axis_index("c")
    s = lax.axis_index("s")
    b = c * (B // 2) + s // 2
    col0 = (s % 2) * HALF

    pltpu.sync_copy(mi_hbm.at[c, s], lbuf)
    pltpu.sync_copy(mf_hbm.at[c, s], fbuf)
    l = lbuf[...][0]
    inv = fbuf[...][0]
    nch = (l + CH - 1) // CH

    def issue(i, buf, sem):
        t0 = i * CH
        t0c = jnp.minimum(t0, L - CH)
        pltpu.make_async_copy(
            xs_hbm.at[b, pl.ds(t0c, CH), pl.ds(col0, HALF)], buf, sem
        ).start()

    def wait(buf, sem):
        pltpu.make_async_copy(
            xs_hbm.at[b, pl.ds(0, CH), pl.ds(col0, HALF)], buf, sem
        ).wait()

    def accum(i, buf, accs):
        t0 = i * CH
        off = t0 - jnp.minimum(t0, L - CH)
        nv = jnp.clip(l - t0, 0, CH)

        def row_body(r, a):
            row = off + r
            return tuple(
                a[v] + buf[row, pl.ds(v * LANES, LANES)] for v in range(NVEC)
            )

        return lax.fori_loop(0, nv, row_body, accs)

    issue(0, buf0, sem0)

    @pl.when(1 < nch)
    def _():
        issue(1, buf1, sem1)

    def pair_body(i2, accs):
        ca = 2 * i2
        wait(buf0, sem0)
        accs = accum(ca, buf0, accs)

        @pl.when(ca + 2 < nch)
        def _():
            issue(ca + 2, buf0, sem0)

        @pl.when(ca + 1 < nch)
        def _():
            wait(buf1, sem1)

        accs = accum(ca + 1, buf1, accs)

        @pl.when(ca + 3 < nch)
        def _():
            issue(ca + 3, buf1, sem1)

        return accs

    accs = tuple(jnp.zeros((LANES,), jnp.float32) for _ in range(NVEC))
    accs = lax.fori_loop(0, (nch + 1) // 2, pair_body, accs)
    for v in range(NVEC):
        obuf[pl.ds(v * LANES, LANES)] = accs[v] * inv
    pltpu.sync_copy(obuf, out_hbm.at[b, pl.ds(col0, HALF)])


@jax.jit
def _mean_pool(xs, mi, mf):
    kern = pl.kernel(
        _body,
        out_type=jax.ShapeDtypeStruct((B, D), jnp.float32),
        mesh=plsc.VectorSubcoreMesh(core_axis_name="c", subcore_axis_name="s"),
        scratch_types=[
            pltpu.VMEM((LANES,), jnp.int32),
            pltpu.VMEM((LANES,), jnp.float32),
            pltpu.VMEM((CH, HALF), jnp.float32),
            pltpu.VMEM((CH, HALF), jnp.float32),
            pltpu.VMEM((HALF,), jnp.float32),
            pltpu.SemaphoreType.DMA,
            pltpu.SemaphoreType.DMA,
        ],
    )
    return kern(xs, mi, mf)


def kernel(xs, xs_len):
    lens = xs_len.astype(jnp.int32)
    inv = 1.0 / lens.astype(jnp.float32)
    cc = jnp.arange(2)[:, None]
    ss = jnp.arange(16)[None, :]
    bmap = cc * (B // 2) + ss // 2
    mi = jnp.broadcast_to(lens[bmap][:, :, None], (2, 16, LANES))
    mf = jnp.broadcast_to(inv[bmap][:, :, None], (2, 16, LANES))
    return _mean_pool(xs, mi, mf)
